# baseline (speedup 1.0000x reference)
import functools

import jax
import jax.numpy as jnp
from jax import lax
from jax.experimental import pallas as pl
from jax.experimental.pallas import tpu as pltpu
from jax.experimental.pallas import tpu_sc as plsc

N, F = 200000, 26
NB, D = 256, 16

NW = 32
CROWS = 500
CELEMS = CROWS * F
NCHUNK = N // CROWS
GROUPS = 63
XBUF = 13312
PAD_X = NCHUNK * 0 + (XBUF - CELEMS)


def _prep_body(x_ref, bins_ref, emb_ref, w_ref, b_ref, bt_ref, t2_ref,
               mn_s, mx_s, *, nsteps):
    i = pl.program_id(0)
    bmin = jnp.min(x_ref[...], axis=0, keepdims=True)
    bmax = jnp.max(x_ref[...], axis=0, keepdims=True)

    @pl.when(i == 0)
    def _():
        mn_s[...] = bmin
        mx_s[...] = bmax

    @pl.when(i > 0)
    def _():
        mn_s[...] = jnp.minimum(mn_s[...], bmin)
        mx_s[...] = jnp.maximum(mx_s[...], bmax)

    @pl.when(i == nsteps - 1)
    def _():
        rng = mx_s[...] - mn_s[...] + 1e-6
        bt_ref[...] = bins_ref[...] * rng + mn_s[...]
        t2 = lax.dot_general(emb_ref[...], w_ref[...],
                             (((1,), (1,)), ((), ())),
                             precision=lax.Precision.HIGHEST,
                             preferred_element_type=jnp.float32)
        t2_ref[...] = t2 + b_ref[...]


def _sc_body(x_hbm, bt_hbm, t2_hbm, out_hbm, btb, t2b, xbuf, posbuf, outbuf):
    wid = lax.axis_index("s") * 2 + lax.axis_index("c")

    pltpu.sync_copy(bt_hbm, btb)
    pltpu.sync_copy(t2_hbm, t2b)

    iota = lax.iota(jnp.int32, 16)
    fvs = [(jnp.int32(jj * 16) + iota) % F for jj in range(13)]

    def chunk_body(ci, _):
        c = wid + NW * ci

        @pl.when(c < NCHUNK)
        def _():
            pltpu.sync_copy(x_hbm.at[pl.ds(c * CELEMS, XBUF)], xbuf)

            def search_group(g, _):
                base = g * 208
                for jj in range(13):
                    off = base + jj * 16
                    xv = xbuf[pl.ds(off, 16)]
                    pos = jnp.zeros((16,), jnp.int32)
                    for s in (128, 64, 32, 16, 8, 4, 2, 1):
                        gidx = (pos + (s - 1)) * F + fvs[jj]
                        bval = plsc.load_gather(btb, [gidx])
                        pos = pos + jnp.where(bval < xv, s, 0)
                    pos = jnp.minimum(pos, NB - 1)
                    posbuf[pl.ds(off, 16)] = pos * D
                return 0

            lax.fori_loop(0, GROUPS, search_group, 0, unroll=False)

            def pool_row(r, _):
                rb = r * F
                v0 = posbuf[pl.ds(rb, 16)]
                v1 = posbuf[pl.ds(rb + 16, 16)]
                acc = t2b[pl.ds(v0[0], D)]
                for f in range(1, F):
                    s = v0[f] if f < 16 else v1[f - 16]
                    acc = acc + t2b[pl.ds(s, D)]
                outbuf[pl.ds(r * D, D)] = acc
                return 0

            lax.fori_loop(0, CROWS, pool_row, 0, unroll=False)

            pltpu.sync_copy(outbuf, out_hbm.at[pl.ds(c * CROWS * D, CROWS * D)])

        return 0

    lax.fori_loop(0, (NCHUNK + NW - 1) // NW, chunk_body, 0, unroll=False)


def kernel(x, bin_logits, embed_table, W, b):
    probs = jax.nn.softmax(bin_logits)
    bins = jnp.cumsum(probs).reshape(NB, 1)

    R1 = 2000
    n1 = N // R1
    bt, t2p = pl.pallas_call(
        functools.partial(_prep_body, nsteps=n1),
        grid=(n1,),
        in_specs=[
            pl.BlockSpec((R1, F), lambda i: (i, 0)),
            pl.BlockSpec((NB, 1), lambda i: (0, 0)),
            pl.BlockSpec((NB, D), lambda i: (0, 0)),
            pl.BlockSpec((D, D), lambda i: (0, 0)),
            pl.BlockSpec((1, D), lambda i: (0, 0)),
        ],
        out_specs=[pl.BlockSpec((NB, F), lambda i: (0, 0)),
                   pl.BlockSpec((NB, D), lambda i: (0, 0))],
        out_shape=[jax.ShapeDtypeStruct((NB, F), jnp.float32),
                   jax.ShapeDtypeStruct((NB, D), jnp.float32)],
        scratch_shapes=[pltpu.VMEM((1, F), jnp.float32),
                        pltpu.VMEM((1, F), jnp.float32)],
        compiler_params=pltpu.CompilerParams(
            dimension_semantics=("arbitrary",)),
    )(x, bins, embed_table, W, b.reshape(1, D))

    x_flat = jnp.pad(x.reshape(-1), (0, XBUF - CELEMS))
    mesh = plsc.VectorSubcoreMesh(core_axis_name="c", subcore_axis_name="s")
    sc_fn = functools.partial(pl.kernel, mesh=mesh,
                              compiler_params=pltpu.CompilerParams(
                                  needs_layout_passes=False),
                              out_type=jax.ShapeDtypeStruct((N * D,),
                                                            jnp.float32),
                              scratch_types=[
                                  pltpu.VMEM((NB * F,), jnp.float32),
                                  pltpu.VMEM((NB * D,), jnp.float32),
                                  pltpu.VMEM((XBUF,), jnp.float32),
                                  pltpu.VMEM((XBUF,), jnp.int32),
                                  pltpu.VMEM((CROWS * D,), jnp.float32),
                              ])(_sc_body)
    out = sc_fn(x_flat, bt.reshape(-1), t2p.reshape(-1))
    return out.reshape(N, D)


# speedup vs baseline: 1.3138x; 1.3138x over previous
import functools

import jax
import jax.numpy as jnp
from jax import lax
from jax.experimental import pallas as pl
from jax.experimental.pallas import tpu as pltpu
from jax.experimental.pallas import tpu_sc as plsc

N, F = 200000, 26
NB, D = 256, 16

NW = 32
CROWS = 512
CELEMS = CROWS * F
NCHUNK = (N + CROWS - 1) // CROWS
NPAD = NCHUNK * CROWS
SGROUPS = CELEMS // 208
PGROUPS = CROWS // 16
MAGIC26 = 20165


def _prep_body(x_ref, bins_ref, emb_ref, w_ref, b_ref, bt_ref, t2_ref,
               mn_s, mx_s, *, nsteps):
    i = pl.program_id(0)
    bmin = jnp.min(x_ref[...], axis=0, keepdims=True)
    bmax = jnp.max(x_ref[...], axis=0, keepdims=True)

    @pl.when(i == 0)
    def _():
        mn_s[...] = bmin
        mx_s[...] = bmax

    @pl.when(i > 0)
    def _():
        mn_s[...] = jnp.minimum(mn_s[...], bmin)
        mx_s[...] = jnp.maximum(mx_s[...], bmax)

    @pl.when(i == nsteps - 1)
    def _():
        rng = mx_s[...] - mn_s[...] + 1e-6
        bt_ref[...] = bins_ref[...] * rng + mn_s[...]
        t2 = lax.dot_general(emb_ref[...], w_ref[...],
                             (((1,), (1,)), ((), ())),
                             precision=lax.Precision.HIGHEST,
                             preferred_element_type=jnp.float32)
        t2_ref[...] = t2 + b_ref[...]


def _sc_body(x_hbm, bt_hbm, t2_hbm, out_hbm, btb, t2b, xbuf, posbuf, outbuf):
    wid = lax.axis_index("s") * 2 + lax.axis_index("c")

    pltpu.sync_copy(bt_hbm, btb)
    pltpu.sync_copy(t2_hbm, t2b)

    iota = lax.iota(jnp.int32, 16)
    fvs = [(jnp.int32(jj * 16) + iota) % F for jj in range(13)]
    iota26 = iota * F
    iota16 = iota * D

    def chunk_body(ci, _):
        c = wid + NW * ci

        @pl.when(c < NCHUNK)
        def _():
            pltpu.sync_copy(x_hbm.at[pl.ds(c * CELEMS, CELEMS)], xbuf)

            def search_group(g, _):
                base = g * 208
                xvs = [xbuf[pl.ds(base + jj * 16, 16)] for jj in range(13)]
                ps = list(fvs)
                for s in (128, 64, 32, 16, 8, 4, 2, 1):
                    c1 = (s - 1) * F
                    step = s * F
                    bvals = [plsc.load_gather(btb, [ps[jj] + c1])
                             for jj in range(13)]
                    for jj in range(13):
                        ps[jj] = ps[jj] + jnp.where(bvals[jj] < xvs[jj],
                                                    step, 0)
                for jj in range(13):
                    pos = jax.lax.shift_right_logical(ps[jj] * MAGIC26, 19)
                    pos16 = jnp.minimum(pos, NB - 1) * D
                    posbuf[pl.ds(base + jj * 16, 16)] = pos16
                return 0

            lax.fori_loop(0, SGROUPS, search_group, 0, unroll=False)

            def pool_group(g, _):
                pb0 = iota26 + g * (16 * F)
                ob0 = iota16 + g * (16 * D)
                pv = plsc.load_gather(posbuf, [pb0])
                accs = [plsc.load_gather(t2b, [pv + d]) for d in range(D)]
                for f in range(1, F):
                    pv = plsc.load_gather(posbuf, [pb0 + f])
                    for d in range(D):
                        accs[d] = accs[d] + plsc.load_gather(t2b, [pv + d])
                for d in range(D):
                    plsc.store_scatter(outbuf, [ob0 + d], accs[d])
                return 0

            lax.fori_loop(0, PGROUPS, pool_group, 0, unroll=False)

            pltpu.sync_copy(outbuf, out_hbm.at[pl.ds(c * CROWS * D, CROWS * D)])

        return 0

    lax.fori_loop(0, (NCHUNK + NW - 1) // NW, chunk_body, 0, unroll=False)


def kernel(x, bin_logits, embed_table, W, b):
    probs = jax.nn.softmax(bin_logits)
    bins = jnp.cumsum(probs).reshape(NB, 1)

    R1 = 2000
    n1 = N // R1
    bt, t2p = pl.pallas_call(
        functools.partial(_prep_body, nsteps=n1),
        grid=(n1,),
        in_specs=[
            pl.BlockSpec((R1, F), lambda i: (i, 0)),
            pl.BlockSpec((NB, 1), lambda i: (0, 0)),
            pl.BlockSpec((NB, D), lambda i: (0, 0)),
            pl.BlockSpec((D, D), lambda i: (0, 0)),
            pl.BlockSpec((1, D), lambda i: (0, 0)),
        ],
        out_specs=[pl.BlockSpec((NB, F), lambda i: (0, 0)),
                   pl.BlockSpec((NB, D), lambda i: (0, 0))],
        out_shape=[jax.ShapeDtypeStruct((NB, F), jnp.float32),
                   jax.ShapeDtypeStruct((NB, D), jnp.float32)],
        scratch_shapes=[pltpu.VMEM((1, F), jnp.float32),
                        pltpu.VMEM((1, F), jnp.float32)],
        compiler_params=pltpu.CompilerParams(
            dimension_semantics=("arbitrary",)),
    )(x, bins, embed_table, W, b.reshape(1, D))

    x_flat = jnp.pad(x.reshape(-1), (0, NPAD * F - N * F))
    mesh = plsc.VectorSubcoreMesh(core_axis_name="c", subcore_axis_name="s")
    sc_fn = functools.partial(pl.kernel, mesh=mesh,
                              compiler_params=pltpu.CompilerParams(
                                  needs_layout_passes=False),
                              out_type=jax.ShapeDtypeStruct((NPAD * D,),
                                                            jnp.float32),
                              scratch_types=[
                                  pltpu.VMEM((NB * F,), jnp.float32),
                                  pltpu.VMEM((NB * D,), jnp.float32),
                                  pltpu.VMEM((CELEMS,), jnp.float32),
                                  pltpu.VMEM((CELEMS,), jnp.int32),
                                  pltpu.VMEM((CROWS * D,), jnp.float32),
                              ])(_sc_body)
    out = sc_fn(x_flat, bt.reshape(-1), t2p.reshape(-1))
    return out[:N * D].reshape(N, D)


# speedup vs baseline: 1.9260x; 1.4660x over previous
import functools

import jax
import jax.numpy as jnp
from jax import lax
from jax.experimental import pallas as pl
from jax.experimental.pallas import tpu as pltpu
from jax.experimental.pallas import tpu_sc as plsc

N, F = 200000, 26
NB, D = 256, 16

NW = 32
CROWS = 512
CELEMS = CROWS * F
NCHUNK = (N + CROWS - 1) // CROWS
NPAD = NCHUNK * CROWS
SGROUPS = CELEMS // 208
PGROUPS = CROWS // 16
MAGIC26 = 20165
DP = D + 1


def _prep_body(x_ref, bins_ref, emb_ref, w_ref, b_ref, bt_ref, t2_ref,
               mn_s, mx_s, *, nsteps):
    i = pl.program_id(0)
    bmin = jnp.min(x_ref[...], axis=0, keepdims=True)
    bmax = jnp.max(x_ref[...], axis=0, keepdims=True)

    @pl.when(i == 0)
    def _():
        mn_s[...] = bmin
        mx_s[...] = bmax

    @pl.when(i > 0)
    def _():
        mn_s[...] = jnp.minimum(mn_s[...], bmin)
        mx_s[...] = jnp.maximum(mx_s[...], bmax)

    @pl.when(i == nsteps - 1)
    def _():
        rng = mx_s[...] - mn_s[...] + 1e-6
        bt_ref[...] = bins_ref[...] * rng + mn_s[...]
        t2 = lax.dot_general(emb_ref[...], w_ref[...],
                             (((1,), (1,)), ((), ())),
                             precision=lax.Precision.HIGHEST,
                             preferred_element_type=jnp.float32)
        t2_ref[...] = jnp.concatenate(
            [t2 + b_ref[...], jnp.zeros((NB, 1), jnp.float32)], axis=1)


def _sc_body(x_hbm, bt_hbm, t2_hbm, out_hbm, btb, t2b, xbuf, posbuf, obuf17,
             outbuf):
    wid = lax.axis_index("s") * 2 + lax.axis_index("c")

    pltpu.sync_copy(bt_hbm, btb)
    pltpu.sync_copy(t2_hbm, t2b)

    iota = lax.iota(jnp.int32, 16)
    fvs = [(jnp.int32(jj * 16) + iota) % F for jj in range(13)]
    iota26 = iota * F

    def chunk_body(ci, _):
        c = wid + NW * ci

        @pl.when(c < NCHUNK)
        def _():
            pltpu.sync_copy(x_hbm.at[pl.ds(c * CELEMS, CELEMS)], xbuf)

            def search_group(g, _):
                base = g * 208
                xvs = [xbuf[pl.ds(base + jj * 16, 16)] for jj in range(13)]
                ps = list(fvs)
                for s in (128, 64, 32, 16, 8, 4, 2, 1):
                    c1 = (s - 1) * F
                    step = s * F
                    bvals = [plsc.load_gather(btb, [ps[jj] + c1])
                             for jj in range(13)]
                    for jj in range(13):
                        ps[jj] = ps[jj] + jnp.where(bvals[jj] < xvs[jj],
                                                    step, 0)
                for jj in range(13):
                    pos = jax.lax.shift_right_logical(ps[jj] * MAGIC26, 19)
                    pos17 = jnp.minimum(pos, NB - 1) * DP
                    posbuf[pl.ds(base + jj * 16, 16)] = pos17
                return 0

            lax.fori_loop(0, SGROUPS, search_group, 0, unroll=False)

            def pool_group(g, _):
                pb0 = iota26 + g * (16 * F)
                ob0 = iota * DP + g * (16 * DP)
                pv = plsc.load_gather(posbuf, [pb0])
                accs = [plsc.load_gather(t2b, [pv + d]) for d in range(D)]
                for f in range(1, F):
                    pv = plsc.load_gather(posbuf, [pb0 + f])
                    for d in range(D):
                        accs[d] = accs[d] + plsc.load_gather(t2b, [pv + d])
                for d in range(D):
                    plsc.store_scatter(obuf17, [ob0 + d], accs[d])
                return 0

            lax.fori_loop(0, PGROUPS, pool_group, 0, unroll=False)

            def repack(r16, _):
                for k in range(16):
                    r = r16 * 16 + k
                    outbuf[pl.ds(r * D, D)] = obuf17[pl.ds(r * DP, D)]
                return 0

            lax.fori_loop(0, CROWS // 16, repack, 0, unroll=False)

            pltpu.sync_copy(outbuf, out_hbm.at[pl.ds(c * CROWS * D, CROWS * D)])

        return 0

    lax.fori_loop(0, (NCHUNK + NW - 1) // NW, chunk_body, 0, unroll=False)


def kernel(x, bin_logits, embed_table, W, b):
    probs = jax.nn.softmax(bin_logits)
    bins = jnp.cumsum(probs).reshape(NB, 1)

    R1 = 2000
    n1 = N // R1
    bt, t2p = pl.pallas_call(
        functools.partial(_prep_body, nsteps=n1),
        grid=(n1,),
        in_specs=[
            pl.BlockSpec((R1, F), lambda i: (i, 0)),
            pl.BlockSpec((NB, 1), lambda i: (0, 0)),
            pl.BlockSpec((NB, D), lambda i: (0, 0)),
            pl.BlockSpec((D, D), lambda i: (0, 0)),
            pl.BlockSpec((1, D), lambda i: (0, 0)),
        ],
        out_specs=[pl.BlockSpec((NB, F), lambda i: (0, 0)),
                   pl.BlockSpec((NB, DP), lambda i: (0, 0))],
        out_shape=[jax.ShapeDtypeStruct((NB, F), jnp.float32),
                   jax.ShapeDtypeStruct((NB, DP), jnp.float32)],
        scratch_shapes=[pltpu.VMEM((1, F), jnp.float32),
                        pltpu.VMEM((1, F), jnp.float32)],
        compiler_params=pltpu.CompilerParams(
            dimension_semantics=("arbitrary",)),
    )(x, bins, embed_table, W, b.reshape(1, D))

    x_flat = jnp.pad(x.reshape(-1), (0, NPAD * F - N * F))
    mesh = plsc.VectorSubcoreMesh(core_axis_name="c", subcore_axis_name="s")
    sc_fn = functools.partial(pl.kernel, mesh=mesh,
                              compiler_params=pltpu.CompilerParams(
                                  needs_layout_passes=False),
                              out_type=jax.ShapeDtypeStruct((NPAD * D,),
                                                            jnp.float32),
                              scratch_types=[
                                  pltpu.VMEM((NB * F,), jnp.float32),
                                  pltpu.VMEM((NB * DP,), jnp.float32),
                                  pltpu.VMEM((CELEMS,), jnp.float32),
                                  pltpu.VMEM((CELEMS,), jnp.int32),
                                  pltpu.VMEM((CROWS * DP,), jnp.float32),
                                  pltpu.VMEM((CROWS * D,), jnp.float32),
                              ])(_sc_body)
    out = sc_fn(x_flat, bt.reshape(-1), t2p.reshape(-1))
    return out[:N * D].reshape(N, D)


# speedup vs baseline: 2.0172x; 1.0474x over previous
import functools

import jax
import jax.numpy as jnp
from jax import lax
from jax.experimental import pallas as pl
from jax.experimental.pallas import tpu as pltpu
from jax.experimental.pallas import tpu_sc as plsc

N, F = 200000, 26
NB, D = 256, 16

NW = 32
CROWS = 400
CELEMS = CROWS * F
NCHUNK = N // CROWS
SGROUPS = CELEMS // 208
PGROUPS = CROWS // 16
MAGIC26 = 20165
DP = D + 1


def _prep_body(x_ref, bins_ref, emb_ref, w_ref, b_ref, bt_ref, t2_ref,
               mn_s, mx_s, *, nsteps):
    i = pl.program_id(0)
    bmin = jnp.min(x_ref[...], axis=0, keepdims=True)
    bmax = jnp.max(x_ref[...], axis=0, keepdims=True)

    @pl.when(i == 0)
    def _():
        mn_s[...] = bmin
        mx_s[...] = bmax

    @pl.when(i > 0)
    def _():
        mn_s[...] = jnp.minimum(mn_s[...], bmin)
        mx_s[...] = jnp.maximum(mx_s[...], bmax)

    @pl.when(i == nsteps - 1)
    def _():
        rng = mx_s[...] - mn_s[...] + 1e-6
        bt_ref[...] = bins_ref[...] * rng + mn_s[...]
        t2 = lax.dot_general(emb_ref[...], w_ref[...],
                             (((1,), (1,)), ((), ())),
                             precision=lax.Precision.HIGHEST,
                             preferred_element_type=jnp.float32)
        t2_ref[...] = jnp.concatenate(
            [t2 + b_ref[...], jnp.zeros((NB, 1), jnp.float32)], axis=1)


def _sc_body(x_hbm, bt_hbm, t2_hbm, out_hbm, btb, t2b, xbuf, posbuf, obuf17,
             outbuf):
    wid = lax.axis_index("s") * 2 + lax.axis_index("c")

    pltpu.sync_copy(bt_hbm, btb)
    pltpu.sync_copy(t2_hbm, t2b)

    iota = lax.iota(jnp.int32, 16)
    fvs = [(jnp.int32(jj * 16) + iota) % F for jj in range(13)]
    iota26 = iota * F

    def chunk_body(ci, _):
        c = wid + NW * ci

        @pl.when(c < NCHUNK)
        def _():
            pltpu.sync_copy(x_hbm.at[pl.ds(c * CELEMS, CELEMS)], xbuf)

            def search_group(g, _):
                base = g * 208
                xvs = [xbuf[pl.ds(base + jj * 16, 16)] for jj in range(13)]
                ps = list(fvs)
                for s in (128, 64, 32, 16, 8, 4, 2, 1):
                    c1 = (s - 1) * F
                    step = s * F
                    bvals = [plsc.load_gather(btb, [ps[jj] + c1])
                             for jj in range(13)]
                    for jj in range(13):
                        ps[jj] = ps[jj] + jnp.where(bvals[jj] < xvs[jj],
                                                    step, 0)
                for jj in range(13):
                    pos = jax.lax.shift_right_logical(ps[jj] * MAGIC26, 19)
                    pos17 = jnp.minimum(pos, NB - 1) * DP
                    posbuf[pl.ds(base + jj * 16, 16)] = pos17
                return 0

            lax.fori_loop(0, SGROUPS, search_group, 0, unroll=False)

            def pool_group(g, _):
                pb0 = iota26 + g * (16 * F)
                ob0 = iota * DP + g * (16 * DP)
                pvs = [plsc.load_gather(posbuf, [pb0 + f]) for f in range(F)]
                accs = [plsc.load_gather(t2b, [pvs[0] + d]) for d in range(D)]
                for f in range(1, F):
                    for d in range(D):
                        accs[d] = accs[d] + plsc.load_gather(t2b,
                                                             [pvs[f] + d])
                for d in range(D):
                    plsc.store_scatter(obuf17, [ob0 + d], accs[d])
                return 0

            lax.fori_loop(0, PGROUPS, pool_group, 0, unroll=False)

            def repack(r16, _):
                for k in range(16):
                    r = r16 * 16 + k
                    outbuf[pl.ds(r * D, D)] = obuf17[pl.ds(r * DP, D)]
                return 0

            lax.fori_loop(0, CROWS // 16, repack, 0, unroll=False)

            pltpu.sync_copy(outbuf, out_hbm.at[pl.ds(c * CROWS * D, CROWS * D)])

        return 0

    lax.fori_loop(0, (NCHUNK + NW - 1) // NW, chunk_body, 0, unroll=False)


def kernel(x, bin_logits, embed_table, W, b):
    probs = jax.nn.softmax(bin_logits)
    bins = jnp.cumsum(probs).reshape(NB, 1)

    R1 = 2000
    n1 = N // R1
    bt, t2p = pl.pallas_call(
        functools.partial(_prep_body, nsteps=n1),
        grid=(n1,),
        in_specs=[
            pl.BlockSpec((R1, F), lambda i: (i, 0)),
            pl.BlockSpec((NB, 1), lambda i: (0, 0)),
            pl.BlockSpec((NB, D), lambda i: (0, 0)),
            pl.BlockSpec((D, D), lambda i: (0, 0)),
            pl.BlockSpec((1, D), lambda i: (0, 0)),
        ],
        out_specs=[pl.BlockSpec((NB, F), lambda i: (0, 0)),
                   pl.BlockSpec((NB, DP), lambda i: (0, 0))],
        out_shape=[jax.ShapeDtypeStruct((NB, F), jnp.float32),
                   jax.ShapeDtypeStruct((NB, DP), jnp.float32)],
        scratch_shapes=[pltpu.VMEM((1, F), jnp.float32),
                        pltpu.VMEM((1, F), jnp.float32)],
        compiler_params=pltpu.CompilerParams(
            dimension_semantics=("arbitrary",)),
    )(x, bins, embed_table, W, b.reshape(1, D))

    x_flat = x.reshape(-1)
    mesh = plsc.VectorSubcoreMesh(core_axis_name="c", subcore_axis_name="s")
    sc_fn = functools.partial(pl.kernel, mesh=mesh,
                              compiler_params=pltpu.CompilerParams(
                                  needs_layout_passes=False),
                              out_type=jax.ShapeDtypeStruct((N * D,),
                                                            jnp.float32),
                              scratch_types=[
                                  pltpu.VMEM((NB * F,), jnp.float32),
                                  pltpu.VMEM((NB * DP,), jnp.float32),
                                  pltpu.VMEM((CELEMS,), jnp.float32),
                                  pltpu.VMEM((CELEMS,), jnp.int32),
                                  pltpu.VMEM((CROWS * DP,), jnp.float32),
                                  pltpu.VMEM((CROWS * D,), jnp.float32),
                              ])(_sc_body)
    out = sc_fn(x_flat, bt.reshape(-1), t2p.reshape(-1))
    return out.reshape(N, D)


# speedup vs baseline: 2.4855x; 1.2322x over previous
import functools

import jax
import jax.numpy as jnp
from jax import lax
from jax.experimental import pallas as pl
from jax.experimental.pallas import tpu as pltpu
from jax.experimental.pallas import tpu_sc as plsc

N, F = 200000, 26
NB, D = 256, 16

NW = 32
CROWS = 400
CELEMS = CROWS * F
NCHUNK = N // CROWS
SGROUPS = CELEMS // 208
PGROUPS = CROWS // 16
MAGIC26 = 20165
DP = D + 1
TP = D // 2 + 1


def _prep_body(x_ref, bins_ref, emb_ref, w_ref, b_ref, bt_ref, t2_ref,
               mn_s, mx_s, *, nsteps):
    i = pl.program_id(0)
    bmin = jnp.min(x_ref[...], axis=0, keepdims=True)
    bmax = jnp.max(x_ref[...], axis=0, keepdims=True)

    @pl.when(i == 0)
    def _():
        mn_s[...] = bmin
        mx_s[...] = bmax

    @pl.when(i > 0)
    def _():
        mn_s[...] = jnp.minimum(mn_s[...], bmin)
        mx_s[...] = jnp.maximum(mx_s[...], bmax)

    @pl.when(i == nsteps - 1)
    def _():
        rng = mx_s[...] - mn_s[...] + 1e-6
        bt_ref[...] = bins_ref[...] * rng + mn_s[...]
        t2 = lax.dot_general(emb_ref[...], w_ref[...],
                             (((1,), (1,)), ((), ())),
                             precision=lax.Precision.HIGHEST,
                             preferred_element_type=jnp.float32)
        tb = t2 + b_ref[...]
        lo = jax.lax.bitcast_convert_type(
            tb[:, :8].astype(jnp.bfloat16), jnp.uint16).astype(jnp.uint32)
        hi = jax.lax.bitcast_convert_type(
            tb[:, 8:].astype(jnp.bfloat16), jnp.uint16).astype(jnp.uint32)
        packed = jax.lax.bitcast_convert_type(lo | (hi << 16), jnp.int32)
        t2_ref[...] = jnp.concatenate(
            [packed, jnp.zeros((NB, 1), jnp.int32)], axis=1)


def _sc_body(x_hbm, bt_hbm, t2_hbm, out_hbm, btb, t2b, xbuf, posbuf, obuf17,
             outbuf):
    wid = lax.axis_index("s") * 2 + lax.axis_index("c")

    pltpu.sync_copy(bt_hbm, btb)
    pltpu.sync_copy(t2_hbm, t2b)

    iota = lax.iota(jnp.int32, 16)
    fvs = [(jnp.int32(jj * 16) + iota) % F for jj in range(13)]
    iota26 = iota * F

    def chunk_body(ci, _):
        c = wid + NW * ci

        @pl.when(c < NCHUNK)
        def _():
            pltpu.sync_copy(x_hbm.at[pl.ds(c * CELEMS, CELEMS)], xbuf)

            def search_group(g, _):
                base = g * 208
                xvs = [xbuf[pl.ds(base + jj * 16, 16)] for jj in range(13)]
                ps = list(fvs)
                for s in (128, 64, 32, 16, 8, 4, 2, 1):
                    c1 = (s - 1) * F
                    step = s * F
                    bvals = [plsc.load_gather(btb, [ps[jj] + c1])
                             for jj in range(13)]
                    for jj in range(13):
                        ps[jj] = ps[jj] + jnp.where(bvals[jj] < xvs[jj],
                                                    step, 0)
                for jj in range(13):
                    pos = jax.lax.shift_right_logical(ps[jj] * MAGIC26, 19)
                    post = jnp.minimum(pos, NB - 1) * TP
                    posbuf[pl.ds(base + jj * 16, 16)] = post
                return 0

            lax.fori_loop(0, SGROUPS, search_group, 0, unroll=False)

            def pool_group(g, _):
                pb0 = iota26 + g * (16 * F)
                ob0 = iota * DP + g * (16 * DP)
                pvs = [plsc.load_gather(posbuf, [pb0 + f]) for f in range(F)]
                himask = jnp.full((16,), -65536, jnp.int32)
                accs = [None] * D
                for f in range(F):
                    for j in range(8):
                        w = plsc.load_gather(t2b, [pvs[f] + j])
                        lo = plsc.bitcast(w << 16, jnp.float32)
                        hi = plsc.bitcast(w & himask, jnp.float32)
                        if f == 0:
                            accs[j], accs[j + 8] = lo, hi
                        else:
                            accs[j] = accs[j] + lo
                            accs[j + 8] = accs[j + 8] + hi
                for d in range(D):
                    plsc.store_scatter(obuf17, [ob0 + d], accs[d])
                return 0

            lax.fori_loop(0, PGROUPS, pool_group, 0, unroll=False)

            def repack(r16, _):
                for k in range(16):
                    r = r16 * 16 + k
                    outbuf[pl.ds(r * D, D)] = obuf17[pl.ds(r * DP, D)]
                return 0

            lax.fori_loop(0, CROWS // 16, repack, 0, unroll=False)

            pltpu.sync_copy(outbuf, out_hbm.at[pl.ds(c * CROWS * D, CROWS * D)])

        return 0

    lax.fori_loop(0, (NCHUNK + NW - 1) // NW, chunk_body, 0, unroll=False)


def kernel(x, bin_logits, embed_table, W, b):
    probs = jax.nn.softmax(bin_logits)
    bins = jnp.cumsum(probs).reshape(NB, 1)

    R1 = 2000
    n1 = N // R1
    bt, t2p = pl.pallas_call(
        functools.partial(_prep_body, nsteps=n1),
        grid=(n1,),
        in_specs=[
            pl.BlockSpec((R1, F), lambda i: (i, 0)),
            pl.BlockSpec((NB, 1), lambda i: (0, 0)),
            pl.BlockSpec((NB, D), lambda i: (0, 0)),
            pl.BlockSpec((D, D), lambda i: (0, 0)),
            pl.BlockSpec((1, D), lambda i: (0, 0)),
        ],
        out_specs=[pl.BlockSpec((NB, F), lambda i: (0, 0)),
                   pl.BlockSpec((NB, TP), lambda i: (0, 0))],
        out_shape=[jax.ShapeDtypeStruct((NB, F), jnp.float32),
                   jax.ShapeDtypeStruct((NB, TP), jnp.int32)],
        scratch_shapes=[pltpu.VMEM((1, F), jnp.float32),
                        pltpu.VMEM((1, F), jnp.float32)],
        compiler_params=pltpu.CompilerParams(
            dimension_semantics=("arbitrary",)),
    )(x, bins, embed_table, W, b.reshape(1, D))

    x_flat = x.reshape(-1)
    mesh = plsc.VectorSubcoreMesh(core_axis_name="c", subcore_axis_name="s")
    sc_fn = functools.partial(pl.kernel, mesh=mesh,
                              compiler_params=pltpu.CompilerParams(
                                  needs_layout_passes=False),
                              out_type=jax.ShapeDtypeStruct((N * D,),
                                                            jnp.float32),
                              scratch_types=[
                                  pltpu.VMEM((NB * F,), jnp.float32),
                                  pltpu.VMEM((NB * TP,), jnp.int32),
                                  pltpu.VMEM((CELEMS,), jnp.float32),
                                  pltpu.VMEM((CELEMS,), jnp.int32),
                                  pltpu.VMEM((CROWS * DP,), jnp.float32),
                                  pltpu.VMEM((CROWS * D,), jnp.float32),
                              ])(_sc_body)
    out = sc_fn(x_flat, bt.reshape(-1), t2p.reshape(-1))
    return out.reshape(N, D)


# speedup vs baseline: 2.6550x; 1.0682x over previous
import functools

import jax
import jax.numpy as jnp
from jax import lax
from jax.experimental import pallas as pl
from jax.experimental.pallas import tpu as pltpu
from jax.experimental.pallas import tpu_sc as plsc

N, F = 200000, 26
NB, D = 256, 16

NW = 32
CROWS = 400
CELEMS = CROWS * F
NCHUNK = N // CROWS
SGROUPS = CELEMS // 208
PGROUPS = CROWS // 16
MAGIC26 = 20165
DP = D + 1
TP = D // 2 + 1


def _prep_body(x_ref, bins_ref, emb_ref, w_ref, b_ref, bt_ref, t2_ref,
               mn_s, mx_s, *, nsteps):
    i = pl.program_id(0)
    bmin = jnp.min(x_ref[...], axis=0, keepdims=True)
    bmax = jnp.max(x_ref[...], axis=0, keepdims=True)

    @pl.when(i == 0)
    def _():
        mn_s[...] = bmin
        mx_s[...] = bmax

    @pl.when(i > 0)
    def _():
        mn_s[...] = jnp.minimum(mn_s[...], bmin)
        mx_s[...] = jnp.maximum(mx_s[...], bmax)

    @pl.when(i == nsteps - 1)
    def _():
        rng = mx_s[...] - mn_s[...] + 1e-6
        bt_ref[...] = bins_ref[...] * rng + mn_s[...]
        t2 = lax.dot_general(emb_ref[...], w_ref[...],
                             (((1,), (1,)), ((), ())),
                             precision=lax.Precision.HIGHEST,
                             preferred_element_type=jnp.float32)
        tb = t2 + b_ref[...]
        lo = jax.lax.bitcast_convert_type(
            tb[:, :8].astype(jnp.bfloat16), jnp.uint16).astype(jnp.uint32)
        hi = jax.lax.bitcast_convert_type(
            tb[:, 8:].astype(jnp.bfloat16), jnp.uint16).astype(jnp.uint32)
        packed = jax.lax.bitcast_convert_type(lo | (hi << 16), jnp.int32)
        t2_ref[...] = jnp.concatenate(
            [packed, jnp.zeros((NB, 1), jnp.int32)], axis=1)


def _sc_body(x_hbm, bt_hbm, t2_hbm, out_hbm, btb, t2b, xbuf, posbuf, obuf17,
             outbuf):
    wid = lax.axis_index("s") * 2 + lax.axis_index("c")

    pltpu.sync_copy(bt_hbm, btb)
    pltpu.sync_copy(t2_hbm, t2b)

    iota = lax.iota(jnp.int32, 16)
    fvs = [(jnp.int32(jj * 16) + iota) % F for jj in range(13)]
    iota26 = iota * F

    def chunk_body(ci, _):
        c = wid + NW * ci

        @pl.when(c < NCHUNK)
        def _():
            pltpu.sync_copy(x_hbm.at[pl.ds(c * CELEMS, CELEMS)], xbuf)

            def search_group(g, _):
                base = g * 208
                xvs = [xbuf[pl.ds(base + jj * 16, 16)] for jj in range(13)]
                ps = list(fvs)
                for s in (128, 64, 32, 16, 8, 4, 2, 1):
                    c1 = (s - 1) * F
                    step = s * F
                    bvals = [plsc.load_gather(btb, [ps[jj] + c1])
                             for jj in range(13)]
                    for jj in range(13):
                        ps[jj] = ps[jj] + jnp.where(bvals[jj] < xvs[jj],
                                                    step, 0)
                for jj in range(13):
                    pos = jax.lax.shift_right_logical(ps[jj] * MAGIC26, 19)
                    post = jnp.minimum(pos, NB - 1) * TP
                    posbuf[pl.ds(base + jj * 16, 16)] = post
                return 0

            lax.fori_loop(0, SGROUPS, search_group, 0, unroll=False)

            def pool_group(g, _):
                pb0 = iota26 + g * (16 * F)
                ob0 = iota * DP + g * (16 * DP)
                pvs = [plsc.load_gather(posbuf, [pb0 + f]) for f in range(F)]
                himask = jnp.full((16,), -65536, jnp.int32)
                accs = [None] * D
                for f in range(F):
                    for j in range(8):
                        w = plsc.load_gather(t2b, [pvs[f] + j])
                        lo = plsc.bitcast(w << 16, jnp.float32)
                        hi = plsc.bitcast(w & himask, jnp.float32)
                        if f == 0:
                            accs[j], accs[j + 8] = lo, hi
                        else:
                            accs[j] = accs[j] + lo
                            accs[j + 8] = accs[j + 8] + hi
                for d in range(D):
                    plsc.store_scatter(obuf17, [ob0 + d], accs[d])
                return 0

            lax.fori_loop(0, PGROUPS, pool_group, 0, unroll=False)

            def repack(r16, _):
                for k in range(16):
                    r = r16 * 16 + k
                    outbuf[pl.ds(r * D, D)] = obuf17[pl.ds(r * DP, D)]
                return 0

            lax.fori_loop(0, CROWS // 16, repack, 0, unroll=False)

            pltpu.sync_copy(outbuf, out_hbm.at[pl.ds(c * CROWS * D, CROWS * D)])

        return 0

    lax.fori_loop(0, (NCHUNK + NW - 1) // NW, chunk_body, 0, unroll=False)


def kernel(x, bin_logits, embed_table, W, b):
    probs = jax.nn.softmax(bin_logits)
    bins = jnp.cumsum(probs).reshape(NB, 1)

    R1 = 8000
    n1 = N // R1
    bt, t2p = pl.pallas_call(
        functools.partial(_prep_body, nsteps=n1),
        grid=(n1,),
        in_specs=[
            pl.BlockSpec((R1, F), lambda i: (i, 0)),
            pl.BlockSpec((NB, 1), lambda i: (0, 0)),
            pl.BlockSpec((NB, D), lambda i: (0, 0)),
            pl.BlockSpec((D, D), lambda i: (0, 0)),
            pl.BlockSpec((1, D), lambda i: (0, 0)),
        ],
        out_specs=[pl.BlockSpec((NB, F), lambda i: (0, 0)),
                   pl.BlockSpec((NB, TP), lambda i: (0, 0))],
        out_shape=[jax.ShapeDtypeStruct((NB, F), jnp.float32),
                   jax.ShapeDtypeStruct((NB, TP), jnp.int32)],
        scratch_shapes=[pltpu.VMEM((1, F), jnp.float32),
                        pltpu.VMEM((1, F), jnp.float32)],
        compiler_params=pltpu.CompilerParams(
            dimension_semantics=("arbitrary",)),
    )(x, bins, embed_table, W, b.reshape(1, D))

    x_flat = x.reshape(-1)
    mesh = plsc.VectorSubcoreMesh(core_axis_name="c", subcore_axis_name="s")
    sc_fn = functools.partial(pl.kernel, mesh=mesh,
                              compiler_params=pltpu.CompilerParams(
                                  needs_layout_passes=False),
                              out_type=jax.ShapeDtypeStruct((N * D,),
                                                            jnp.float32),
                              scratch_types=[
                                  pltpu.VMEM((NB * F,), jnp.float32),
                                  pltpu.VMEM((NB * TP,), jnp.int32),
                                  pltpu.VMEM((CELEMS,), jnp.float32),
                                  pltpu.VMEM((CELEMS,), jnp.int32),
                                  pltpu.VMEM((CROWS * DP,), jnp.float32),
                                  pltpu.VMEM((CROWS * D,), jnp.float32),
                              ])(_sc_body)
    out = sc_fn(x_flat, bt.reshape(-1), t2p.reshape(-1))
    return out.reshape(N, D)


# speedup vs baseline: 2.6795x; 1.0093x over previous
import functools

import jax
import jax.numpy as jnp
from jax import lax
from jax.experimental import pallas as pl
from jax.experimental.pallas import tpu as pltpu
from jax.experimental.pallas import tpu_sc as plsc

N, F = 200000, 26
NB, D = 256, 16

NW = 32
CROWS = 400
CELEMS = CROWS * F
NCHUNK = N // CROWS
SGROUPS = CELEMS // 208
PGROUPS = CROWS // 16
MAGIC26 = 20165
DP = D + 1
TP = D // 2 + 1


def _prep_body(x_ref, bins_ref, emb_ref, w_ref, b_ref, bt_ref, t2_ref,
               mn_s, mx_s, *, nsteps):
    i = pl.program_id(0)
    bmin = jnp.min(x_ref[...], axis=0, keepdims=True)
    bmax = jnp.max(x_ref[...], axis=0, keepdims=True)

    @pl.when(i == 0)
    def _():
        mn_s[...] = bmin
        mx_s[...] = bmax

    @pl.when(i > 0)
    def _():
        mn_s[...] = jnp.minimum(mn_s[...], bmin)
        mx_s[...] = jnp.maximum(mx_s[...], bmax)

    @pl.when(i == nsteps - 1)
    def _():
        rng = mx_s[...] - mn_s[...] + 1e-6
        bt_ref[...] = bins_ref[...] * rng + mn_s[...]
        t2 = lax.dot_general(emb_ref[...], w_ref[...],
                             (((1,), (1,)), ((), ())),
                             precision=lax.Precision.HIGHEST,
                             preferred_element_type=jnp.float32)
        tb = t2 + b_ref[...]
        lo = jax.lax.bitcast_convert_type(
            tb[:, :8].astype(jnp.bfloat16), jnp.uint16).astype(jnp.uint32)
        hi = jax.lax.bitcast_convert_type(
            tb[:, 8:].astype(jnp.bfloat16), jnp.uint16).astype(jnp.uint32)
        packed = jax.lax.bitcast_convert_type(lo | (hi << 16), jnp.int32)
        t2_ref[...] = jnp.concatenate(
            [packed, jnp.zeros((NB, 1), jnp.int32)], axis=1)


def _sc_body(x_hbm, bt_hbm, t2_hbm, out_hbm, btb, t2b, xbuf, posbuf, obuf17,
             outbuf):
    wid = lax.axis_index("s") * 2 + lax.axis_index("c")

    pltpu.sync_copy(bt_hbm, btb)
    pltpu.sync_copy(t2_hbm, t2b)

    iota = lax.iota(jnp.int32, 16)
    fvs = [(jnp.int32(jj * 16) + iota) % F for jj in range(13)]
    iota26 = iota * F

    def chunk_body(ci, _):
        c = wid + NW * ci

        @pl.when(c < NCHUNK)
        def _():
            pltpu.sync_copy(x_hbm.at[pl.ds(c * CELEMS, CELEMS)], xbuf)

            def search_group(g, _):
                base = g * 208
                xvs = [xbuf[pl.ds(base + jj * 16, 16)] for jj in range(13)]
                ps = list(fvs)
                for s in (128, 64, 32, 16, 8, 4, 2, 1):
                    c1 = (s - 1) * F
                    step = s * F
                    bvals = [plsc.load_gather(btb, [ps[jj] + c1])
                             for jj in range(13)]
                    for jj in range(13):
                        ps[jj] = ps[jj] + jnp.where(bvals[jj] < xvs[jj],
                                                    step, 0)
                for jj in range(13):
                    pos = jax.lax.shift_right_logical(ps[jj] * MAGIC26, 19)
                    post = jnp.minimum(pos, NB - 1) * TP
                    posbuf[pl.ds(base + jj * 16, 16)] = post
                return 0

            lax.fori_loop(0, SGROUPS, search_group, 0, unroll=False)

            def pool_group(g, _):
                pb0 = iota26 + g * (16 * F)
                ob0 = iota * DP + g * (16 * DP)
                pvs = [plsc.load_gather(posbuf, [pb0 + f]) for f in range(F)]
                himask = jnp.full((16,), -65536, jnp.int32)
                accs = [None] * D
                for f in range(F):
                    for j in range(8):
                        w = plsc.load_gather(t2b, [pvs[f] + j])
                        lo = plsc.bitcast(w << 16, jnp.float32)
                        hi = plsc.bitcast(w & himask, jnp.float32)
                        if f == 0:
                            accs[j], accs[j + 8] = lo, hi
                        else:
                            accs[j] = accs[j] + lo
                            accs[j + 8] = accs[j + 8] + hi
                for d in range(D):
                    plsc.store_scatter(obuf17, [ob0 + d], accs[d])
                return 0

            lax.fori_loop(0, PGROUPS, pool_group, 0, unroll=False)

            def repack(r16, _):
                for k in range(16):
                    r = r16 * 16 + k
                    outbuf[pl.ds(r * D, D)] = obuf17[pl.ds(r * DP, D)]
                return 0

            lax.fori_loop(0, CROWS // 16, repack, 0, unroll=False)

            pltpu.sync_copy(outbuf, out_hbm.at[pl.ds(c * CROWS * D, CROWS * D)])

        return 0

    lax.fori_loop(0, (NCHUNK + NW - 1) // NW, chunk_body, 0, unroll=False)


def kernel(x, bin_logits, embed_table, W, b):
    probs = jax.nn.softmax(bin_logits)
    bins = jnp.cumsum(probs).reshape(NB, 1)

    R1 = 20000
    n1 = N // R1
    bt, t2p = pl.pallas_call(
        functools.partial(_prep_body, nsteps=n1),
        grid=(n1,),
        in_specs=[
            pl.BlockSpec((R1, F), lambda i: (i, 0)),
            pl.BlockSpec((NB, 1), lambda i: (0, 0)),
            pl.BlockSpec((NB, D), lambda i: (0, 0)),
            pl.BlockSpec((D, D), lambda i: (0, 0)),
            pl.BlockSpec((1, D), lambda i: (0, 0)),
        ],
        out_specs=[pl.BlockSpec((NB, F), lambda i: (0, 0)),
                   pl.BlockSpec((NB, TP), lambda i: (0, 0))],
        out_shape=[jax.ShapeDtypeStruct((NB, F), jnp.float32),
                   jax.ShapeDtypeStruct((NB, TP), jnp.int32)],
        scratch_shapes=[pltpu.VMEM((1, F), jnp.float32),
                        pltpu.VMEM((1, F), jnp.float32)],
        compiler_params=pltpu.CompilerParams(
            dimension_semantics=("arbitrary",)),
    )(x, bins, embed_table, W, b.reshape(1, D))

    x_flat = x.reshape(-1)
    mesh = plsc.VectorSubcoreMesh(core_axis_name="c", subcore_axis_name="s")
    sc_fn = functools.partial(pl.kernel, mesh=mesh,
                              compiler_params=pltpu.CompilerParams(
                                  needs_layout_passes=False),
                              out_type=jax.ShapeDtypeStruct((N * D,),
                                                            jnp.float32),
                              scratch_types=[
                                  pltpu.VMEM((NB * F,), jnp.float32),
                                  pltpu.VMEM((NB * TP,), jnp.int32),
                                  pltpu.VMEM((CELEMS,), jnp.float32),
                                  pltpu.VMEM((CELEMS,), jnp.int32),
                                  pltpu.VMEM((CROWS * DP,), jnp.float32),
                                  pltpu.VMEM((CROWS * D,), jnp.float32),
                              ])(_sc_body)
    out = sc_fn(x_flat, bt.reshape(-1), t2p.reshape(-1))
    return out.reshape(N, D)
